# baseline (speedup 1.0000x reference)
import functools

import jax
import jax.numpy as jnp
from jax import lax
from jax.experimental import pallas as pl
from jax.experimental.pallas import tpu as pltpu
from jax.experimental.pallas import tpu_sc as plsc

B = 16384
D = 64
L = 16
NC = 2
NS = 16
NW = NC * NS
N_ENT = 1000000
WD = 2 * D

EBLK = 128
SUP = 256
NSUP = N_ENT // SUP
NSUP_BASE = NSUP // NW
NSUP_EXTRA = NSUP % NW
TAIL = N_ENT - NSUP * SUP
TAIL0 = NSUP * SUP

ROWS_PER_W = B // NW
CHUNK = 128
NCHUNK = ROWS_PER_W // CHUNK


def _repack_body(re_t, im_t, tail_cat, cat_out,
                 inb0, inb1, outb0, outb1,
                 sem_i0, sem_i1, sem_o0, sem_o1):
    wid = lax.axis_index("s") * NC + lax.axis_index("c")
    lane = lax.iota(jnp.int32, L)
    sup0 = wid * NSUP_BASE + jnp.minimum(wid, NSUP_EXTRA)
    inbs = (inb0, inb1)
    outbs = (outb0, outb1)
    sems_i = (sem_i0, sem_i1)
    sems_o = (sem_o0, sem_o1)

    def in_issue(s, inb, sem):
        e0 = s * SUP
        for j in range(D // 8):
            pltpu.async_copy(re_t.at[pl.ds(8 * j, 8), pl.ds(e0, SUP)],
                             inb.at[pl.ds(8 * j, 8)], sem)
            pltpu.async_copy(im_t.at[pl.ds(8 * j, 8), pl.ds(e0, SUP)],
                             inb.at[pl.ds(D + 8 * j, 8)], sem)

    def in_wait(inb, sem):
        for _ in range(2 * (D // 8)):
            pltpu.make_async_copy(re_t.at[pl.ds(0, 8), pl.ds(0, SUP)],
                                  inb.at[pl.ds(0, 8)], sem).wait()

    def out_wait(q):
        pltpu.make_async_copy(outbs[q], cat_out.at[pl.ds(0, EBLK)],
                              sems_o[q]).wait()

    def shuffle(inb, half, q):
        outb = outbs[q]

        @plsc.parallel_loop(0, WD, unroll=8)
        def col(c0):
            cols = (c0 + lane) & (WD - 1)
            for g in range(EBLK // L):
                rows = g * L + lane
                v = plsc.load_gather(inb, [cols, half * EBLK + rows])
                plsc.store_scatter(outb, [rows, cols], v)

    in_issue(sup0, inbs[0], sems_i[0])
    in_issue(sup0 + 1, inbs[1], sems_i[1])

    def pair(i, carry):
        for p in (0, 1):
            s = sup0 + 2 * i + p
            in_wait(inbs[p], sems_i[p])
            for half in (0, 1):
                q = half
                if p == 0:
                    @pl.when(i > 0)
                    def _():
                        out_wait(q)
                else:
                    out_wait(q)
                shuffle(inbs[p], half, q)
                pltpu.async_copy(
                    outbs[q],
                    cat_out.at[pl.ds(s * SUP + half * EBLK, EBLK)],
                    sems_o[q])

            @pl.when(2 * i + p + 2 < NSUP_BASE)
            def _():
                in_issue(s + 2, inbs[p], sems_i[p])
        return carry

    lax.fori_loop(0, NSUP_BASE // 2, pair, 0)
    out_wait(0)
    out_wait(1)

    @pl.when(wid < NSUP_EXTRA)
    def _():
        s = sup0 + NSUP_BASE
        in_issue(s, inbs[0], sems_i[0])
        in_wait(inbs[0], sems_i[0])
        for half in (0, 1):
            shuffle(inbs[0], half, half)
            pltpu.async_copy(
                outbs[half],
                cat_out.at[pl.ds(s * SUP + half * EBLK, EBLK)],
                sems_o[half])
        out_wait(0)
        out_wait(1)

    @pl.when(wid == NW - 1)
    def _():
        pltpu.sync_copy(tail_cat, inb0.at[pl.ds(0, TAIL), pl.ds(0, WD)])
        pltpu.sync_copy(inb0.at[pl.ds(0, TAIL), pl.ds(0, WD)],
                        cat_out.at[pl.ds(TAIL0, TAIL)])


def _lookup_body(h_hbm, r_hbm, t_hbm, ent_cat, rel_cat, out_hbm,
                 h_idx, r_idx, t_idx, hb, tb, rb, scores, sem0, sem1):
    wid = lax.axis_index("s") * NC + lax.axis_index("c")
    base = wid * ROWS_PER_W

    pltpu.sync_copy(h_hbm.at[pl.ds(base, ROWS_PER_W)], h_idx)
    pltpu.sync_copy(r_hbm.at[pl.ds(base, ROWS_PER_W)], r_idx)
    pltpu.sync_copy(t_hbm.at[pl.ds(base, ROWS_PER_W)], t_idx)

    sems = (sem0, sem1)

    def fire(j, slot):
        sem = sems[slot]
        return [
            pltpu.async_copy(ent_cat.at[h_idx.at[pl.ds(j * CHUNK, CHUNK)]],
                             hb.at[slot], sem),
            pltpu.async_copy(ent_cat.at[t_idx.at[pl.ds(j * CHUNK, CHUNK)]],
                             tb.at[slot], sem),
            pltpu.async_copy(rel_cat.at[r_idx.at[pl.ds(j * CHUNK, CHUNK)]],
                             rb.at[slot], sem),
        ]

    def compute(j, slot):
        hrow = hb.at[slot]
        trow = tb.at[slot]
        rrow = rb.at[slot]

        @plsc.parallel_loop(0, CHUNK // L, unroll=2)
        def group(g):
            lane = lax.iota(jnp.int32, L)
            rows = g * L + lane
            acc = jnp.zeros((L,), jnp.float32)
            for c in range(D):
                rot = (lane + c) & (D - 1)
                rot_im = rot + D
                a_re = plsc.load_gather(hrow, [rows, rot])
                a_im = plsc.load_gather(hrow, [rows, rot_im])
                b_re = plsc.load_gather(trow, [rows, rot])
                b_im = plsc.load_gather(trow, [rows, rot_im])
                c_re = plsc.load_gather(rrow, [rows, rot])
                c_im = plsc.load_gather(rrow, [rows, rot_im])
                acc = acc + c_re * (a_re * b_re + a_im * b_im) \
                          + c_im * (a_re * b_im - a_im * b_re)
            scores[pl.ds(j * CHUNK + g * L, L)] = acc

    handles = {0: fire(0, 0)}
    for j in range(NCHUNK):
        if j + 1 < NCHUNK:
            handles[j + 1] = fire(j + 1, (j + 1) % 2)
        for hd in handles.pop(j):
            hd.wait()
        compute(j, j % 2)

    pltpu.sync_copy(scores, out_hbm.at[pl.ds(base, ROWS_PER_W)])


def _mesh():
    return plsc.VectorSubcoreMesh(core_axis_name="c", subcore_axis_name="s")


def _repack(ent_re_t, ent_im_t, tail_cat):
    run = functools.partial(
        pl.kernel,
        mesh=_mesh(),
        compiler_params=pltpu.CompilerParams(
            needs_layout_passes=False, use_tc_tiling_on_sc=True),
        out_type=jax.ShapeDtypeStruct((N_ENT, WD), jnp.float32),
        scratch_types=[
            pltpu.VMEM((WD, SUP), jnp.float32),
            pltpu.VMEM((WD, SUP), jnp.float32),
            pltpu.VMEM((EBLK, WD), jnp.float32),
            pltpu.VMEM((EBLK, WD), jnp.float32),
            pltpu.SemaphoreType.DMA,
            pltpu.SemaphoreType.DMA,
            pltpu.SemaphoreType.DMA,
            pltpu.SemaphoreType.DMA,
        ],
    )(_repack_body)
    return run(ent_re_t, ent_im_t, tail_cat)


def _lookup(h, r, t, ent_cat, rel_cat):
    run = functools.partial(
        pl.kernel,
        mesh=_mesh(),
        compiler_params=pltpu.CompilerParams(
            needs_layout_passes=False, use_tc_tiling_on_sc=True),
        out_type=jax.ShapeDtypeStruct((B,), jnp.float32),
        scratch_types=[
            pltpu.VMEM((ROWS_PER_W,), jnp.int32),
            pltpu.VMEM((ROWS_PER_W,), jnp.int32),
            pltpu.VMEM((ROWS_PER_W,), jnp.int32),
            pltpu.VMEM((2, CHUNK, WD), jnp.float32),
            pltpu.VMEM((2, CHUNK, WD), jnp.float32),
            pltpu.VMEM((2, CHUNK, WD), jnp.float32),
            pltpu.VMEM((ROWS_PER_W,), jnp.float32),
            pltpu.SemaphoreType.DMA,
            pltpu.SemaphoreType.DMA,
        ],
    )(_lookup_body)
    return run(h, r, t, ent_cat, rel_cat)


def kernel(h, r, t, ent_re, ent_im, rel_re, rel_im):
    tail_cat = jnp.concatenate(
        [ent_re[TAIL0:], ent_im[TAIL0:]], axis=1)
    ent_cat = _repack(ent_re.T, ent_im.T, tail_cat)
    rel_cat = jnp.concatenate([rel_re, rel_im], axis=1)
    return _lookup(h.astype(jnp.int32), r.astype(jnp.int32),
                   t.astype(jnp.int32), ent_cat, rel_cat)


# speedup vs baseline: 1.0020x; 1.0020x over previous
import functools

import jax
import jax.numpy as jnp
from jax import lax
from jax.experimental import pallas as pl
from jax.experimental.pallas import tpu as pltpu
from jax.experimental.pallas import tpu_sc as plsc

B = 16384
D = 64
L = 16
NC = 2
NS = 16
NW = NC * NS
N_ENT = 1000000
WD = 2 * D

EBLK = 128
SUP = 256
NSUP = N_ENT // SUP
NSUP_BASE = NSUP // NW
NSUP_EXTRA = NSUP % NW
TAIL = N_ENT - NSUP * SUP
TAIL0 = NSUP * SUP

ROWS_PER_W = B // NW
CHUNK = 128
NCHUNK = ROWS_PER_W // CHUNK


def _repack_body(re_t, im_t, tail_cat, cat_out,
                 inb0, inb1, outb0, outb1,
                 sem_i0, sem_i1, sem_o0, sem_o1):
    wid = lax.axis_index("s") * NC + lax.axis_index("c")
    lane = lax.iota(jnp.int32, L)
    sup0 = wid * NSUP_BASE + jnp.minimum(wid, NSUP_EXTRA)
    inbs = (inb0, inb1)
    outbs = (outb0, outb1)
    sems_i = (sem_i0, sem_i1)
    sems_o = (sem_o0, sem_o1)

    def in_issue(s, inb, sem):
        e0 = s * SUP
        for j in range(D // 8):
            pltpu.async_copy(re_t.at[pl.ds(8 * j, 8), pl.ds(e0, SUP)],
                             inb.at[pl.ds(8 * j, 8)], sem)
            pltpu.async_copy(im_t.at[pl.ds(8 * j, 8), pl.ds(e0, SUP)],
                             inb.at[pl.ds(D + 8 * j, 8)], sem)

    def in_wait(inb, sem):
        for _ in range(2 * (D // 8)):
            pltpu.make_async_copy(re_t.at[pl.ds(0, 8), pl.ds(0, SUP)],
                                  inb.at[pl.ds(0, 8)], sem).wait()

    def out_wait(q):
        pltpu.make_async_copy(outbs[q], cat_out.at[pl.ds(0, EBLK)],
                              sems_o[q]).wait()

    def shuffle(inb, half, q):
        outb = outbs[q]

        @plsc.parallel_loop(0, WD, unroll=4)
        def col(c0):
            cols = (c0 + lane) & (WD - 1)
            for g in range(EBLK // L):
                rows = g * L + lane
                v = plsc.load_gather(inb, [cols, half * EBLK + rows])
                plsc.store_scatter(outb, [rows, cols], v)

    in_issue(sup0, inbs[0], sems_i[0])
    in_issue(sup0 + 1, inbs[1], sems_i[1])

    def pair(i, carry):
        for p in (0, 1):
            s = sup0 + 2 * i + p
            in_wait(inbs[p], sems_i[p])
            for half in (0, 1):
                q = half
                if p == 0:
                    @pl.when(i > 0)
                    def _():
                        out_wait(q)
                else:
                    out_wait(q)
                shuffle(inbs[p], half, q)
                pltpu.async_copy(
                    outbs[q],
                    cat_out.at[pl.ds(s * SUP + half * EBLK, EBLK)],
                    sems_o[q])

            @pl.when(2 * i + p + 2 < NSUP_BASE)
            def _():
                in_issue(s + 2, inbs[p], sems_i[p])
        return carry

    lax.fori_loop(0, NSUP_BASE // 2, pair, 0)
    out_wait(0)
    out_wait(1)

    @pl.when(wid < NSUP_EXTRA)
    def _():
        s = sup0 + NSUP_BASE
        in_issue(s, inbs[0], sems_i[0])
        in_wait(inbs[0], sems_i[0])
        for half in (0, 1):
            shuffle(inbs[0], half, half)
            pltpu.async_copy(
                outbs[half],
                cat_out.at[pl.ds(s * SUP + half * EBLK, EBLK)],
                sems_o[half])
        out_wait(0)
        out_wait(1)

    @pl.when(wid == NW - 1)
    def _():
        pltpu.sync_copy(tail_cat, inb0.at[pl.ds(0, TAIL), pl.ds(0, WD)])
        pltpu.sync_copy(inb0.at[pl.ds(0, TAIL), pl.ds(0, WD)],
                        cat_out.at[pl.ds(TAIL0, TAIL)])


def _lookup_body(h_hbm, r_hbm, t_hbm, ent_cat, rel_cat, out_hbm,
                 h_idx, r_idx, t_idx, hb, tb, rb, scores, sem0, sem1):
    wid = lax.axis_index("s") * NC + lax.axis_index("c")
    base = wid * ROWS_PER_W

    pltpu.sync_copy(h_hbm.at[pl.ds(base, ROWS_PER_W)], h_idx)
    pltpu.sync_copy(r_hbm.at[pl.ds(base, ROWS_PER_W)], r_idx)
    pltpu.sync_copy(t_hbm.at[pl.ds(base, ROWS_PER_W)], t_idx)

    sems = (sem0, sem1)

    def fire(j, slot):
        sem = sems[slot]
        return [
            pltpu.async_copy(ent_cat.at[h_idx.at[pl.ds(j * CHUNK, CHUNK)]],
                             hb.at[slot], sem),
            pltpu.async_copy(ent_cat.at[t_idx.at[pl.ds(j * CHUNK, CHUNK)]],
                             tb.at[slot], sem),
            pltpu.async_copy(rel_cat.at[r_idx.at[pl.ds(j * CHUNK, CHUNK)]],
                             rb.at[slot], sem),
        ]

    def compute(j, slot):
        hrow = hb.at[slot]
        trow = tb.at[slot]
        rrow = rb.at[slot]

        @plsc.parallel_loop(0, CHUNK // L, unroll=2)
        def group(g):
            lane = lax.iota(jnp.int32, L)
            rows = g * L + lane
            acc = jnp.zeros((L,), jnp.float32)
            for c in range(D):
                rot = (lane + c) & (D - 1)
                rot_im = rot + D
                a_re = plsc.load_gather(hrow, [rows, rot])
                a_im = plsc.load_gather(hrow, [rows, rot_im])
                b_re = plsc.load_gather(trow, [rows, rot])
                b_im = plsc.load_gather(trow, [rows, rot_im])
                c_re = plsc.load_gather(rrow, [rows, rot])
                c_im = plsc.load_gather(rrow, [rows, rot_im])
                acc = acc + c_re * (a_re * b_re + a_im * b_im) \
                          + c_im * (a_re * b_im - a_im * b_re)
            scores[pl.ds(j * CHUNK + g * L, L)] = acc

    handles = {0: fire(0, 0)}
    for j in range(NCHUNK):
        if j + 1 < NCHUNK:
            handles[j + 1] = fire(j + 1, (j + 1) % 2)
        for hd in handles.pop(j):
            hd.wait()
        compute(j, j % 2)

    pltpu.sync_copy(scores, out_hbm.at[pl.ds(base, ROWS_PER_W)])


def _mesh():
    return plsc.VectorSubcoreMesh(core_axis_name="c", subcore_axis_name="s")


def _repack(ent_re_t, ent_im_t, tail_cat):
    run = functools.partial(
        pl.kernel,
        mesh=_mesh(),
        compiler_params=pltpu.CompilerParams(
            needs_layout_passes=False, use_tc_tiling_on_sc=True),
        out_type=jax.ShapeDtypeStruct((N_ENT, WD), jnp.float32),
        scratch_types=[
            pltpu.VMEM((WD, SUP), jnp.float32),
            pltpu.VMEM((WD, SUP), jnp.float32),
            pltpu.VMEM((EBLK, WD), jnp.float32),
            pltpu.VMEM((EBLK, WD), jnp.float32),
            pltpu.SemaphoreType.DMA,
            pltpu.SemaphoreType.DMA,
            pltpu.SemaphoreType.DMA,
            pltpu.SemaphoreType.DMA,
        ],
    )(_repack_body)
    return run(ent_re_t, ent_im_t, tail_cat)


def _lookup(h, r, t, ent_cat, rel_cat):
    run = functools.partial(
        pl.kernel,
        mesh=_mesh(),
        compiler_params=pltpu.CompilerParams(
            needs_layout_passes=False, use_tc_tiling_on_sc=True),
        out_type=jax.ShapeDtypeStruct((B,), jnp.float32),
        scratch_types=[
            pltpu.VMEM((ROWS_PER_W,), jnp.int32),
            pltpu.VMEM((ROWS_PER_W,), jnp.int32),
            pltpu.VMEM((ROWS_PER_W,), jnp.int32),
            pltpu.VMEM((2, CHUNK, WD), jnp.float32),
            pltpu.VMEM((2, CHUNK, WD), jnp.float32),
            pltpu.VMEM((2, CHUNK, WD), jnp.float32),
            pltpu.VMEM((ROWS_PER_W,), jnp.float32),
            pltpu.SemaphoreType.DMA,
            pltpu.SemaphoreType.DMA,
        ],
    )(_lookup_body)
    return run(h, r, t, ent_cat, rel_cat)


def kernel(h, r, t, ent_re, ent_im, rel_re, rel_im):
    tail_cat = jnp.concatenate(
        [ent_re[TAIL0:], ent_im[TAIL0:]], axis=1)
    ent_cat = _repack(ent_re.T, ent_im.T, tail_cat)
    rel_cat = jnp.concatenate([rel_re, rel_im], axis=1)
    return _lookup(h.astype(jnp.int32), r.astype(jnp.int32),
                   t.astype(jnp.int32), ent_cat, rel_cat)


# speedup vs baseline: 1.0773x; 1.0751x over previous
import functools

import jax
import jax.numpy as jnp
from jax import lax
from jax.experimental import pallas as pl
from jax.experimental.pallas import tpu as pltpu
from jax.experimental.pallas import tpu_sc as plsc

B = 16384
D = 64
L = 16
NC = 2
NS = 16
NW = NC * NS
N_ENT = 1000000
WD = 2 * D

EBLK = 128
SUP = 256
NSUP = N_ENT // SUP
NSUP_BASE = NSUP // NW
NSUP_EXTRA = NSUP % NW
TAIL = N_ENT - NSUP * SUP
TAIL0 = NSUP * SUP

ROWS_PER_W = B // NW
CHUNK = 128
NCHUNK = ROWS_PER_W // CHUNK


def _repack_body(re_t, im_t, tail_cat, cat_out,
                 inb0, inb1, outb0, outb1,
                 sem_i0, sem_i1, sem_o0, sem_o1):
    wid = lax.axis_index("s") * NC + lax.axis_index("c")
    lane = lax.iota(jnp.int32, L)
    sup0 = wid * NSUP_BASE + jnp.minimum(wid, NSUP_EXTRA)
    inbs = (inb0, inb1)
    outbs = (outb0, outb1)
    sems_i = (sem_i0, sem_i1)
    sems_o = (sem_o0, sem_o1)

    def in_issue(s, inb, sem):
        e0 = s * SUP
        for j in range(D // 8):
            pltpu.async_copy(re_t.at[pl.ds(8 * j, 8), pl.ds(e0, SUP)],
                             inb.at[pl.ds(8 * j, 8)], sem)
            pltpu.async_copy(im_t.at[pl.ds(8 * j, 8), pl.ds(e0, SUP)],
                             inb.at[pl.ds(D + 8 * j, 8)], sem)

    def in_wait(inb, sem):
        for _ in range(2 * (D // 8)):
            pltpu.make_async_copy(re_t.at[pl.ds(0, 8), pl.ds(0, SUP)],
                                  inb.at[pl.ds(0, 8)], sem).wait()

    def out_wait(q):
        pltpu.make_async_copy(outbs[q], cat_out.at[pl.ds(0, EBLK)],
                              sems_o[q]).wait()

    def shuffle(inb, half, q):
        outb = outbs[q]

        @plsc.parallel_loop(0, WD, unroll=4)
        def col(c0):
            cols = (c0 + lane) & (WD - 1)
            for g in range(EBLK // L):
                rows = g * L + lane
                v = plsc.load_gather(inb, [cols, half * EBLK + rows])
                plsc.store_scatter(outb, [rows, cols], v)

    in_issue(sup0, inbs[0], sems_i[0])
    in_issue(sup0 + 1, inbs[1], sems_i[1])

    def pair(i, carry):
        for p in (0, 1):
            s = sup0 + 2 * i + p
            in_wait(inbs[p], sems_i[p])
            for half in (0, 1):
                q = half
                if p == 0:
                    @pl.when(i > 0)
                    def _():
                        out_wait(q)
                else:
                    out_wait(q)
                shuffle(inbs[p], half, q)
                pltpu.async_copy(
                    outbs[q],
                    cat_out.at[pl.ds(s * SUP + half * EBLK, EBLK)],
                    sems_o[q])

            @pl.when(2 * i + p + 2 < NSUP_BASE)
            def _():
                in_issue(s + 2, inbs[p], sems_i[p])
        return carry

    lax.fori_loop(0, NSUP_BASE // 2, pair, 0)
    out_wait(0)
    out_wait(1)

    @pl.when(wid < NSUP_EXTRA)
    def _():
        s = sup0 + NSUP_BASE
        in_issue(s, inbs[0], sems_i[0])
        in_wait(inbs[0], sems_i[0])
        for half in (0, 1):
            shuffle(inbs[0], half, half)
            pltpu.async_copy(
                outbs[half],
                cat_out.at[pl.ds(s * SUP + half * EBLK, EBLK)],
                sems_o[half])
        out_wait(0)
        out_wait(1)

    @pl.when(wid == NW - 1)
    def _():
        pltpu.sync_copy(tail_cat, inb0.at[pl.ds(0, TAIL), pl.ds(0, WD)])
        pltpu.sync_copy(inb0.at[pl.ds(0, TAIL), pl.ds(0, WD)],
                        cat_out.at[pl.ds(TAIL0, TAIL)])


def _lookup_body(h_hbm, r_hbm, t_hbm, ent_cat, rel_cat, out_hbm,
                 h_idx, r_idx, t_idx, hb, tb, rb, scores, sem0, sem1):
    wid = lax.axis_index("s") * NC + lax.axis_index("c")
    base = wid * ROWS_PER_W

    pltpu.sync_copy(h_hbm.at[pl.ds(base, ROWS_PER_W)], h_idx)
    pltpu.sync_copy(r_hbm.at[pl.ds(base, ROWS_PER_W)], r_idx)
    pltpu.sync_copy(t_hbm.at[pl.ds(base, ROWS_PER_W)], t_idx)

    sems = (sem0, sem1)

    def fire(j, slot):
        sem = sems[slot]
        return [
            pltpu.async_copy(ent_cat.at[h_idx.at[pl.ds(j * CHUNK, CHUNK)]],
                             hb.at[slot], sem),
            pltpu.async_copy(ent_cat.at[t_idx.at[pl.ds(j * CHUNK, CHUNK)]],
                             tb.at[slot], sem),
            pltpu.async_copy(rel_cat.at[r_idx.at[pl.ds(j * CHUNK, CHUNK)]],
                             rb.at[slot], sem),
        ]

    def compute(j, slot):
        hrow = hb.at[slot]
        trow = tb.at[slot]
        rrow = rb.at[slot]

        def group(g, carry):
            lane = lax.iota(jnp.int32, L)
            rows = g * L + lane
            acc = jnp.zeros((L,), jnp.float32)
            for c in range(D):
                rot = (lane + c) & (D - 1)
                rot_im = rot + D
                a_re = plsc.load_gather(hrow, [rows, rot])
                a_im = plsc.load_gather(hrow, [rows, rot_im])
                b_re = plsc.load_gather(trow, [rows, rot])
                b_im = plsc.load_gather(trow, [rows, rot_im])
                c_re = plsc.load_gather(rrow, [rows, rot])
                c_im = plsc.load_gather(rrow, [rows, rot_im])
                acc = acc + c_re * (a_re * b_re + a_im * b_im) \
                          + c_im * (a_re * b_im - a_im * b_re)
            scores[pl.ds(j * CHUNK + g * L, L)] = acc
            return carry

        lax.fori_loop(0, CHUNK // L, group, 0)

    handles = {0: fire(0, 0)}
    for j in range(NCHUNK):
        if j + 1 < NCHUNK:
            handles[j + 1] = fire(j + 1, (j + 1) % 2)
        for hd in handles.pop(j):
            hd.wait()
        compute(j, j % 2)

    pltpu.sync_copy(scores, out_hbm.at[pl.ds(base, ROWS_PER_W)])


def _mesh():
    return plsc.VectorSubcoreMesh(core_axis_name="c", subcore_axis_name="s")


def _repack(ent_re_t, ent_im_t, tail_cat):
    run = functools.partial(
        pl.kernel,
        mesh=_mesh(),
        compiler_params=pltpu.CompilerParams(
            needs_layout_passes=False, use_tc_tiling_on_sc=True),
        out_type=jax.ShapeDtypeStruct((N_ENT, WD), jnp.float32),
        scratch_types=[
            pltpu.VMEM((WD, SUP), jnp.float32),
            pltpu.VMEM((WD, SUP), jnp.float32),
            pltpu.VMEM((EBLK, WD), jnp.float32),
            pltpu.VMEM((EBLK, WD), jnp.float32),
            pltpu.SemaphoreType.DMA,
            pltpu.SemaphoreType.DMA,
            pltpu.SemaphoreType.DMA,
            pltpu.SemaphoreType.DMA,
        ],
    )(_repack_body)
    return run(ent_re_t, ent_im_t, tail_cat)


def _lookup(h, r, t, ent_cat, rel_cat):
    run = functools.partial(
        pl.kernel,
        mesh=_mesh(),
        compiler_params=pltpu.CompilerParams(
            needs_layout_passes=False, use_tc_tiling_on_sc=True),
        out_type=jax.ShapeDtypeStruct((B,), jnp.float32),
        scratch_types=[
            pltpu.VMEM((ROWS_PER_W,), jnp.int32),
            pltpu.VMEM((ROWS_PER_W,), jnp.int32),
            pltpu.VMEM((ROWS_PER_W,), jnp.int32),
            pltpu.VMEM((2, CHUNK, WD), jnp.float32),
            pltpu.VMEM((2, CHUNK, WD), jnp.float32),
            pltpu.VMEM((2, CHUNK, WD), jnp.float32),
            pltpu.VMEM((ROWS_PER_W,), jnp.float32),
            pltpu.SemaphoreType.DMA,
            pltpu.SemaphoreType.DMA,
        ],
    )(_lookup_body)
    return run(h, r, t, ent_cat, rel_cat)


def kernel(h, r, t, ent_re, ent_im, rel_re, rel_im):
    tail_cat = jnp.concatenate(
        [ent_re[TAIL0:], ent_im[TAIL0:]], axis=1)
    ent_cat = _repack(ent_re.T, ent_im.T, tail_cat)
    rel_cat = jnp.concatenate([rel_re, rel_im], axis=1)
    return _lookup(h.astype(jnp.int32), r.astype(jnp.int32),
                   t.astype(jnp.int32), ent_cat, rel_cat)
